# baseline (speedup 1.0000x reference)
import functools
import jax
import jax.numpy as jnp
from jax import lax
from jax.experimental import pallas as pl
from jax.experimental.pallas import tpu as pltpu
from jax.experimental.pallas import tpu_sc as plsc

NBUF = 5
HALVES = 1


def _make_kernel(B, S, D):
    info = plsc.get_sparse_core_info()
    NC, NS = info.num_cores, info.num_subcores
    NW = NC * NS
    assert B % NW == 0
    cols = B // NW
    hc = cols // HALVES
    n_chunks = S * HALVES
    assert n_chunks % NBUF == 0
    n_outer = n_chunks // NBUF

    mesh = plsc.VectorSubcoreMesh(core_axis_name="c", subcore_axis_name="s")

    @functools.partial(
        pl.kernel,
        mesh=mesh,
        compiler_params=pltpu.CompilerParams(
            disable_bounds_checks=True,
            disable_semaphore_checks=True,
            skip_device_barrier=True,
        ),
        out_type=jax.ShapeDtypeStruct((S, B, D), jnp.float32),
        scratch_types=(
            [pltpu.VMEM((S, cols), jnp.int32)]
            + [pltpu.VMEM((hc, D), jnp.float32) for _ in range(NBUF)]
            + [pltpu.SemaphoreType.DMA, pltpu.SemaphoreType.DMA]
        ),
    )
    def k(table_hbm, xt_hbm, out_hbm, idx_v, *rest):
        bufs = rest[:NBUF]
        sem_g, sem_w = rest[NBUF], rest[NBUF + 1]
        wid = lax.axis_index("s") * NC + lax.axis_index("c")
        b0 = wid * cols

        def chunk_idx(c):
            s = c // HALVES
            h = c % HALVES
            return s, h * hc

        pltpu.sync_copy(xt_hbm.at[:, pl.ds(b0, cols)], idx_v)

        def gather(c, buf):
            s, co = chunk_idx(c)
            return pltpu.async_copy(
                table_hbm.at[idx_v.at[s, pl.ds(co, hc)]], buf, sem_g
            )

        def wait_gather(c, buf):
            s, co = chunk_idx(c)
            pltpu.make_async_copy(
                table_hbm.at[idx_v.at[s, pl.ds(co, hc)]], buf, sem_g
            ).wait()

        def write(c, buf):
            s, co = chunk_idx(c)
            return pltpu.async_copy(
                buf, out_hbm.at[s, pl.ds(b0 + co, hc)], sem_w
            )

        def wait_write(c, buf):
            s, co = chunk_idx(c)
            pltpu.make_async_copy(
                buf, out_hbm.at[s, pl.ds(b0 + co, hc)], sem_w
            ).wait()

        def outer(o, carry):
            s0 = o * NBUF
            for b in range(NBUF):
                @pl.when(o > 0)
                def _():
                    wait_write(s0 - NBUF + b, bufs[b])

                gather(s0 + b, bufs[b])
            for b in range(NBUF):
                wait_gather(s0 + b, bufs[b])
                write(s0 + b, bufs[b])
            return carry

        lax.fori_loop(0, n_outer, outer, 0)
        for b in range(NBUF):
            wait_write((n_outer - 1) * NBUF + b, bufs[b])

    return k


def kernel(x, weight):
    B, S = x.shape
    V, D = weight.shape
    k = _make_kernel(B, S, D)
    xt = jnp.transpose(x.astype(jnp.int32), (1, 0))
    out3 = k(weight, xt)
    return jnp.transpose(out3, (1, 0, 2))


# speedup vs baseline: 1.0023x; 1.0023x over previous
import functools
import jax
import jax.numpy as jnp
from jax import lax
from jax.experimental import pallas as pl
from jax.experimental.pallas import tpu as pltpu
from jax.experimental.pallas import tpu_sc as plsc

NBUF = 5
HALVES = 1


def _make_kernel(B, S, D):
    info = plsc.get_sparse_core_info()
    NC, NS = info.num_cores, info.num_subcores
    NW = NC * NS
    assert B % NW == 0
    cols = B // NW
    hc = cols // HALVES
    n_chunks = S * HALVES
    assert n_chunks % NBUF == 0
    n_outer = n_chunks // NBUF

    mesh = plsc.VectorSubcoreMesh(core_axis_name="c", subcore_axis_name="s")

    @functools.partial(
        pl.kernel,
        mesh=mesh,
        out_type=jax.ShapeDtypeStruct((S, B, D), jnp.float32),
        scratch_types=(
            [pltpu.VMEM((S, cols), jnp.int32)]
            + [pltpu.VMEM((hc, D), jnp.float32) for _ in range(NBUF)]
            + [pltpu.SemaphoreType.DMA, pltpu.SemaphoreType.DMA]
        ),
    )
    def k(table_hbm, xt_hbm, out_hbm, idx_v, *rest):
        bufs = rest[:NBUF]
        sem_g, sem_w = rest[NBUF], rest[NBUF + 1]
        wid = lax.axis_index("s") * NC + lax.axis_index("c")
        b0 = wid * cols

        def chunk_idx(c):
            s = c // HALVES
            h = c % HALVES
            return s, h * hc

        pltpu.sync_copy(xt_hbm.at[:, pl.ds(b0, cols)], idx_v)

        def gather(c, buf):
            s, co = chunk_idx(c)
            return pltpu.async_copy(
                table_hbm.at[idx_v.at[s, pl.ds(co, hc)]], buf, sem_g
            )

        def wait_gather(c, buf):
            s, co = chunk_idx(c)
            pltpu.make_async_copy(
                table_hbm.at[idx_v.at[s, pl.ds(co, hc)]], buf, sem_g
            ).wait()

        def write(c, buf):
            s, co = chunk_idx(c)
            return pltpu.async_copy(
                buf, out_hbm.at[s, pl.ds(b0 + co, hc)], sem_w
            )

        def wait_write(c, buf):
            s, co = chunk_idx(c)
            pltpu.make_async_copy(
                buf, out_hbm.at[s, pl.ds(b0 + co, hc)], sem_w
            ).wait()

        def outer(o, carry):
            s0 = o * NBUF
            for b in range(NBUF):
                @pl.when(o > 0)
                def _():
                    wait_write(s0 - NBUF + b, bufs[b])

                gather(s0 + b, bufs[b])
            for b in range(NBUF):
                wait_gather(s0 + b, bufs[b])
                write(s0 + b, bufs[b])
            return carry

        lax.fori_loop(0, n_outer, outer, 0)
        for b in range(NBUF):
            wait_write((n_outer - 1) * NBUF + b, bufs[b])

    return k


def kernel(x, weight):
    B, S = x.shape
    V, D = weight.shape
    k = _make_kernel(B, S, D)
    xt = jnp.transpose(x.astype(jnp.int32), (1, 0))
    out3 = k(weight, xt)
    return jnp.transpose(out3, (1, 0, 2))
